# Initial kernel scaffold; baseline (speedup 1.0000x reference)
import jax
import jax.numpy as jnp
from jax.experimental import pallas as pl


def kernel(x, emb, W, b):
    raise NotImplementedError("write your pallas kernel here")


# speedup vs baseline: 1.2297x; 1.2297x over previous
import functools

import jax
import jax.numpy as jnp
from jax import lax
from jax.experimental import pallas as pl
from jax.experimental.pallas import tpu as pltpu
from jax.experimental.pallas import tpu_sc as plsc

_NUM_EMB = 16909
_EMB_DIM = 200
_OUT_DIM = 512
_BATCH = 4
_SEQ = 16906
_B_TOT = _BATCH * _SEQ


_BM = 512


def _proj_body(emb_ref, w_ref, b_ref, out_ref):
    out_ref[...] = (
        jnp.dot(emb_ref[...], w_ref[...], preferred_element_type=jnp.float32)
        + b_ref[...]
    )


def _project(emb, w, b):
    return pl.pallas_call(
        _proj_body,
        grid=(pl.cdiv(_NUM_EMB, _BM),),
        in_specs=[
            pl.BlockSpec((_BM, _EMB_DIM), lambda i: (i, 0)),
            pl.BlockSpec((_EMB_DIM, _OUT_DIM), lambda i: (0, 0)),
            pl.BlockSpec((1, _OUT_DIM), lambda i: (0, 0)),
        ],
        out_specs=pl.BlockSpec((_BM, _OUT_DIM), lambda i: (i, 0)),
        out_shape=jax.ShapeDtypeStruct((_NUM_EMB, _OUT_DIM), jnp.float32),
    )(emb, w, b.reshape(1, _OUT_DIM))


_NW = 32
_WROWS = 128
_CHUNK = 2120
_LAST_CHUNK = _B_TOT - 31 * _CHUNK
_MAXWIN = 17
_NWIN_LAST = 15

_mesh = plsc.VectorSubcoreMesh(core_axis_name="c", subcore_axis_name="s")


@functools.partial(
    pl.kernel,
    out_type=jax.ShapeDtypeStruct((_B_TOT, _OUT_DIM), jnp.float32),
    mesh=_mesh,
    scratch_types=[
        pltpu.VMEM((_WROWS,), jnp.int32),
        pltpu.VMEM((_WROWS, _OUT_DIM), jnp.float32),
        pltpu.SemaphoreType.DMA,
    ],
)
def _gather(table_hbm, idx_hbm, out_hbm, idx_v, rows_v, sem):
    wid = lax.axis_index("s") * 2 + lax.axis_index("c")
    base = wid * _CHUNK
    chunk = jnp.where(wid < _NW - 1, _CHUNK, _LAST_CHUNK)
    nwin = jnp.where(wid < _NW - 1, _MAXWIN, _NWIN_LAST)

    @pl.loop(0, _MAXWIN)
    def _win(i):
        @pl.when(i < nwin)
        def _():
            off = jnp.minimum(i * _WROWS, chunk - _WROWS)
            start = base + off
            pltpu.sync_copy(idx_hbm.at[pl.ds(start, _WROWS)], idx_v)

            @pl.loop(0, _WROWS, step=16)
            def _norm(j):
                v = idx_v[pl.ds(j, 16)]
                idx_v[pl.ds(j, 16)] = lax.rem(v + _NUM_EMB, _NUM_EMB)

            pltpu.async_copy(table_hbm.at[idx_v], rows_v, sem).wait()
            pltpu.sync_copy(rows_v, out_hbm.at[pl.ds(start, _WROWS)])


def kernel(x, emb, W, b):
    proj = _project(emb, W, b)
    flat = _gather(proj, x.reshape(_B_TOT))
    return flat.reshape(_BATCH, _SEQ, _OUT_DIM)


# speedup vs baseline: 1.2830x; 1.0433x over previous
import functools

import jax
import jax.numpy as jnp
from jax import lax
from jax.experimental import pallas as pl
from jax.experimental.pallas import tpu as pltpu
from jax.experimental.pallas import tpu_sc as plsc

_NUM_EMB = 16909
_EMB_DIM = 200
_OUT_DIM = 512
_BATCH = 4
_SEQ = 16906
_B_TOT = _BATCH * _SEQ


_BM = 512


def _proj_body(emb_ref, w_ref, b_ref, out_ref):
    out_ref[...] = (
        jnp.dot(emb_ref[...], w_ref[...], preferred_element_type=jnp.float32)
        + b_ref[...]
    )


def _project(emb, w, b):
    return pl.pallas_call(
        _proj_body,
        grid=(pl.cdiv(_NUM_EMB, _BM),),
        in_specs=[
            pl.BlockSpec((_BM, _EMB_DIM), lambda i: (i, 0)),
            pl.BlockSpec((_EMB_DIM, _OUT_DIM), lambda i: (0, 0)),
            pl.BlockSpec((1, _OUT_DIM), lambda i: (0, 0)),
        ],
        out_specs=pl.BlockSpec((_BM, _OUT_DIM), lambda i: (i, 0)),
        out_shape=jax.ShapeDtypeStruct((_NUM_EMB, _OUT_DIM), jnp.float32),
    )(emb, w, b.reshape(1, _OUT_DIM))


_NW = 32
_WROWS = 112
_CHUNK = 2120
_LAST_CHUNK = _B_TOT - 31 * _CHUNK
_MAXWIN = 19
_NWIN_LAST = 17

_mesh = plsc.VectorSubcoreMesh(core_axis_name="c", subcore_axis_name="s")


@functools.partial(
    pl.kernel,
    out_type=jax.ShapeDtypeStruct((_B_TOT, _OUT_DIM), jnp.float32),
    mesh=_mesh,
    scratch_types=[
        pltpu.VMEM((2, _WROWS), jnp.int32),
        pltpu.VMEM((2, _WROWS, _OUT_DIM), jnp.float32),
        pltpu.SemaphoreType.DMA,
        pltpu.SemaphoreType.DMA,
        pltpu.SemaphoreType.DMA,
        pltpu.SemaphoreType.DMA,
    ],
)
def _gather(table_hbm, idx_hbm, out_hbm, idx2, rows2, g0, g1, w0, w1):
    gsem = (g0, g1)
    wsem = (w0, w1)
    wid = lax.axis_index("s") * 2 + lax.axis_index("c")
    base = wid * _CHUNK
    chunk = jnp.where(wid < _NW - 1, _CHUNK, _LAST_CHUNK)
    nwin = jnp.where(wid < _NW - 1, _MAXWIN, _NWIN_LAST)

    def win_start(i):
        return base + jnp.minimum(i * _WROWS, chunk - _WROWS)

    def start_gather(i, s):
        start = win_start(i)
        pltpu.sync_copy(idx_hbm.at[pl.ds(start, _WROWS)], idx2.at[s])
        for j in range(0, _WROWS, 16):
            v = idx2[s, pl.ds(j, 16)]
            idx2[s, pl.ds(j, 16)] = lax.rem(v + _NUM_EMB, _NUM_EMB)
        pltpu.async_copy(table_hbm.at[idx2.at[s]], rows2.at[s], gsem[s])

    def wait_gather(s):
        pltpu.make_async_copy(table_hbm.at[idx2.at[s]], rows2.at[s],
                              gsem[s]).wait()

    def start_wb(i, s):
        pltpu.async_copy(rows2.at[s], out_hbm.at[pl.ds(win_start(i), _WROWS)],
                         wsem[s])

    def wait_wb(s):
        pltpu.make_async_copy(rows2.at[s], out_hbm.at[pl.ds(0, _WROWS)],
                              wsem[s]).wait()

    @pl.loop(0, (_MAXWIN + 1) // 2)
    def _pair(k):
        a = 2 * k
        b = a + 1

        @pl.when(a < nwin)
        def _():
            @pl.when(k > 0)
            def _():
                wait_wb(0)
            start_gather(a, 0)

        @pl.when((b - 2 >= 0) & (b - 2 < nwin))
        def _():
            wait_gather(1)
            start_wb(b - 2, 1)

        @pl.when(b < nwin)
        def _():
            @pl.when(k > 0)
            def _():
                wait_wb(1)
            start_gather(b, 1)

        @pl.when(a < nwin)
        def _():
            wait_gather(0)
            start_wb(a, 0)

    wait_wb(0)
    wait_wb(1)


def kernel(x, emb, W, b):
    proj = _project(emb, W, b)
    flat = _gather(proj, x.reshape(_B_TOT))
    return flat.reshape(_BATCH, _SEQ, _OUT_DIM)
